# Initial kernel scaffold; baseline (speedup 1.0000x reference)
import jax
import jax.numpy as jnp
from jax.experimental import pallas as pl


def kernel(x):
    raise NotImplementedError("write your pallas kernel here")


# speedup vs baseline: 1.5887x; 1.5887x over previous
import jax
import jax.numpy as jnp
from jax.experimental import pallas as pl
from jax.experimental.pallas import tpu as pltpu

_K = 32
_N = 8192
_R = 8


def _kmax_block(x_ref, o_ref):
    vals = x_ref[...]
    r = vals.shape[0]
    lane = jax.lax.broadcasted_iota(jnp.int32, (r, _N), 1)
    neg = jnp.float32(-jnp.inf)
    sel_vals = []
    sel_idx = []
    big = jnp.int32(_N)
    for _ in range(_K):
        m = jnp.max(vals, axis=-1, keepdims=True)
        i = jnp.min(jnp.where(vals == m, lane, big), axis=-1)
        sel_vals.append(m[:, 0])
        sel_idx.append(i)
        vals = jnp.where(lane == i[:, None], neg, vals)
    sel = jnp.stack(sel_vals, axis=-1)
    idx = jnp.stack(sel_idx, axis=-1)
    rank = jnp.sum((idx[:, None, :] < idx[:, :, None]).astype(jnp.int32),
                   axis=-1)
    cols = []
    for j in range(_K):
        w = rank == j
        cols.append(jnp.sum(jnp.where(w, sel, 0.0), axis=-1))
    o_ref[...] = jnp.stack(cols, axis=-1)


def kernel(x):
    b0, b1, n = x.shape
    rows = b0 * b1
    xr = x.reshape(rows, n)
    out = pl.pallas_call(
        _kmax_block,
        grid=(rows // _R,),
        in_specs=[pl.BlockSpec((_R, n), lambda i: (i, 0))],
        out_specs=pl.BlockSpec((_R, _K), lambda i: (i, 0)),
        out_shape=jax.ShapeDtypeStruct((rows, _K), jnp.float32),
    )(xr)
    return out.reshape(b0, b1, _K)


# speedup vs baseline: 8.6466x; 5.4425x over previous
import functools

import jax
import jax.numpy as jnp
from jax import lax
from jax.experimental import pallas as pl
from jax.experimental.pallas import tpu as pltpu
from jax.experimental.pallas import tpu_sc as plsc

_K = 32
_N = 8192
_NV = _N // 16
_ROWS = 16384
_NW = 32
_RPW = _ROWS // _NW
_RB = 2
_NB = _RPW // _RB

_NEG = float("-inf")


def _sort_desc(v):
    s, _ = plsc.sort_key_val(v, v, descending=True)
    return s


def _rev(v):
    return lax.rev(v, dimensions=(0,))


def _merge32_16(t0, t1, b):
    n1 = _sort_desc(jnp.maximum(t1, _rev(b)))
    hi = jnp.maximum(t0, _rev(n1))
    lo = jnp.minimum(t0, _rev(n1))
    return _sort_desc(hi), _sort_desc(lo)


def _process_row(buf, roff, cand, outbuf, out_base):
    lane = lax.iota(jnp.int32, 16)
    ninfv = jnp.full((16,), _NEG, jnp.float32)
    zero = jnp.zeros((16,), jnp.int32)
    one = jnp.full((16,), 1, jnp.int32)

    def fold_body(g, acc):
        f0, f1 = acc
        base = roff + g * 256
        xs = [buf[pl.ds(base + i * 16, 16)] for i in range(16)]

        def tree8(v):
            m01 = jnp.maximum(v[0], v[1])
            m23 = jnp.maximum(v[2], v[3])
            m45 = jnp.maximum(v[4], v[5])
            m67 = jnp.maximum(v[6], v[7])
            return jnp.maximum(jnp.maximum(m01, m23), jnp.maximum(m45, m67))

        return jnp.maximum(f0, tree8(xs[:8])), jnp.maximum(f1, tree8(xs[8:]))

    f0, f1 = lax.fori_loop(0, _N // 256, fold_body, (ninfv, ninfv))
    t = jnp.min(jnp.minimum(f0, f1))
    tv = jnp.broadcast_to(t, (16,))

    def scan_body(v, cnt):
        xv = buf[pl.ds(roff + v * 16, 16)]
        m = xv >= tv
        pc = jnp.cumsum(jnp.where(m, one, zero))
        pos = cnt + pc - 1
        plsc.store_scatter(cand, [pos], xv, mask=m)
        return cnt + plsc.all_reduce_population_count(m)

    cntv = lax.fori_loop(0, _NV, scan_body, zero)
    n = jnp.max(cntv)

    c0 = _sort_desc(cand[pl.ds(0, 16)])
    c1 = _sort_desc(cand[pl.ds(16, 16)])
    hi = jnp.maximum(c0, _rev(c1))
    lo = jnp.minimum(c0, _rev(c1))
    top = (_sort_desc(hi), _sort_desc(lo))
    nv = (n + 15) // 16

    def merge_body(vi, acc):
        b = cand[pl.ds(vi * 16, 16)]
        valid = (vi * 16 + lane) < n
        b = jnp.where(valid, b, ninfv)
        return _merge32_16(acc[0], acc[1], _sort_desc(b))

    t0, t1 = lax.fori_loop(2, nv, merge_body, top)
    T = jnp.min(t1)
    Tv = jnp.broadcast_to(T, (16,))

    def strict_body(vi, acc):
        b = cand[pl.ds(vi * 16, 16)]
        valid = (vi * 16 + lane) < n
        m = (b > Tv) & valid
        return acc + plsc.all_reduce_population_count(m)

    nstrict = lax.fori_loop(0, nv, strict_body, zero)
    needv = 32 - nstrict

    def keep_body(vi, acc):
        eqrun, keeprun = acc
        b = cand[pl.ds(vi * 16, 16)]
        valid = (vi * 16 + lane) < n
        strict = (b > Tv) & valid
        eq = (b == Tv) & valid
        eqpc = eqrun + jnp.cumsum(jnp.where(eq, one, zero))
        keep = strict | (eq & (eqpc <= needv))
        kpc = jnp.cumsum(jnp.where(keep, one, zero))
        pos = out_base + keeprun + kpc - 1
        plsc.store_scatter(outbuf, [pos], b, mask=keep)
        return (eqrun + plsc.all_reduce_population_count(eq),
                keeprun + plsc.all_reduce_population_count(keep))

    lax.fori_loop(0, nv, keep_body, (zero, zero))


_mesh = plsc.VectorSubcoreMesh(core_axis_name="c", subcore_axis_name="s")


@functools.partial(
    pl.kernel,
    out_type=jax.ShapeDtypeStruct((_ROWS * _K,), jnp.float32),
    mesh=_mesh,
    compiler_params=pltpu.CompilerParams(needs_layout_passes=False),
    scratch_types=[
        pltpu.VMEM((_RB * _N,), jnp.float32),
        pltpu.VMEM((_RB * _N,), jnp.float32),
        pltpu.VMEM((_N,), jnp.float32),
        pltpu.VMEM((_RPW * _K,), jnp.float32),
        pltpu.SemaphoreType.DMA,
        pltpu.SemaphoreType.DMA,
    ],
)
def _sc_kmax(x_hbm, out_hbm, buf0, buf1, cand, outbuf, sem0, sem1):
    nc = 2
    wid = lax.axis_index("s") * nc + lax.axis_index("c")
    ebase = wid * _RPW * _N

    def start(buf, sem, batch):
        pltpu.async_copy(
            x_hbm.at[pl.ds(ebase + batch * (_RB * _N), _RB * _N)], buf, sem)

    def wait(buf, sem, batch):
        pltpu.make_async_copy(
            x_hbm.at[pl.ds(ebase + batch * (_RB * _N), _RB * _N)], buf,
            sem).wait()

    start(buf0, sem0, 0)
    start(buf1, sem1, 1)

    def outer(i, carry):
        b0 = i * 2
        wait(buf0, sem0, b0)
        for r in range(_RB):
            _process_row(buf0, r * _N, cand, outbuf, (b0 * _RB + r) * _K)

        @pl.when(b0 + 2 < _NB)
        def _():
            start(buf0, sem0, b0 + 2)

        b1 = b0 + 1
        wait(buf1, sem1, b1)
        for r in range(_RB):
            _process_row(buf1, r * _N, cand, outbuf, (b1 * _RB + r) * _K)

        @pl.when(b1 + 2 < _NB)
        def _():
            start(buf1, sem1, b1 + 2)

        return carry

    lax.fori_loop(0, _NB // 2, outer, 0)
    pltpu.sync_copy(outbuf, out_hbm.at[pl.ds(wid * (_RPW * _K), _RPW * _K)])


def kernel(x):
    b0, b1, n = x.shape
    xr = x.reshape(b0 * b1 * n)
    flat = _sc_kmax(xr)
    return flat.reshape(b0, b1, _K)


# speedup vs baseline: 9.6990x; 1.1217x over previous
import functools

import jax
import jax.numpy as jnp
from jax import lax
from jax.experimental import pallas as pl
from jax.experimental.pallas import tpu as pltpu
from jax.experimental.pallas import tpu_sc as plsc

_K = 32
_N = 8192
_NV = _N // 16
_ROWS = 16384
_NW = 32
_RPW = _ROWS // _NW
_RB = 2
_NB = _RPW // _RB

_NEG = float("-inf")


def _sort_desc(v):
    s, _ = plsc.sort_key_val(v, v, descending=True)
    return s


def _rev(v):
    return lax.rev(v, dimensions=(0,))


def _merge32_16(t0, t1, b):
    n1 = _sort_desc(jnp.maximum(t1, _rev(b)))
    hi = jnp.maximum(t0, _rev(n1))
    lo = jnp.minimum(t0, _rev(n1))
    return _sort_desc(hi), _sort_desc(lo)


def _process_row(buf, roff, cand, outbuf, out_base):
    lane = lax.iota(jnp.int32, 16)
    ninfv = jnp.full((16,), _NEG, jnp.float32)
    zero = jnp.zeros((16,), jnp.int32)
    one = jnp.full((16,), 1, jnp.int32)

    def fold_body(g, acc):
        f0, f1 = acc
        base = roff + g * 256
        xs = [buf[pl.ds(base + i * 16, 16)] for i in range(16)]

        def tree8(v):
            m01 = jnp.maximum(v[0], v[1])
            m23 = jnp.maximum(v[2], v[3])
            m45 = jnp.maximum(v[4], v[5])
            m67 = jnp.maximum(v[6], v[7])
            return jnp.maximum(jnp.maximum(m01, m23), jnp.maximum(m45, m67))

        return jnp.maximum(f0, tree8(xs[:8])), jnp.maximum(f1, tree8(xs[8:]))

    f0, f1 = lax.fori_loop(0, _N // 256, fold_body, (ninfv, ninfv))
    t = jnp.min(jnp.minimum(f0, f1))
    tv = jnp.broadcast_to(t, (16,))

    def scan_body(v, cnt_s):
        xv = buf[pl.ds(roff + v * 16, 16)]
        m = xv >= tv
        plsc.store_compressed(cand.at[pl.ds(cnt_s, 16)], xv, mask=m)
        return cnt_s + plsc.all_reduce_population_count(m)[0]

    n = lax.fori_loop(0, _NV, scan_body, jnp.int32(0))

    c0 = _sort_desc(cand[pl.ds(0, 16)])
    c1 = _sort_desc(cand[pl.ds(16, 16)])
    hi = jnp.maximum(c0, _rev(c1))
    lo = jnp.minimum(c0, _rev(c1))
    top = (_sort_desc(hi), _sort_desc(lo))
    nv = (n + 15) // 16

    def merge_body(vi, acc):
        b = cand[pl.ds(vi * 16, 16)]
        valid = (vi * 16 + lane) < n
        b = jnp.where(valid, b, ninfv)
        return _merge32_16(acc[0], acc[1], _sort_desc(b))

    t0, t1 = lax.fori_loop(2, nv, merge_body, top)
    T = jnp.min(t1)
    Tv = jnp.broadcast_to(T, (16,))

    def strict_body(vi, acc):
        b = cand[pl.ds(vi * 16, 16)]
        valid = (vi * 16 + lane) < n
        m = (b > Tv) & valid
        return acc + plsc.all_reduce_population_count(m)

    nstrict = lax.fori_loop(0, nv, strict_body, zero)
    needv = 32 - nstrict

    def keep_body(vi, acc):
        eqrun, keeprun = acc
        b = cand[pl.ds(vi * 16, 16)]
        valid = (vi * 16 + lane) < n
        strict = (b > Tv) & valid
        eq = (b == Tv) & valid
        eqpc = eqrun + jnp.cumsum(jnp.where(eq, one, zero))
        keep = strict | (eq & (eqpc <= needv))
        kpc = jnp.cumsum(jnp.where(keep, one, zero))
        pos = out_base + keeprun + kpc - 1
        plsc.store_scatter(outbuf, [pos], b, mask=keep)
        return (eqrun + plsc.all_reduce_population_count(eq),
                keeprun + plsc.all_reduce_population_count(keep))

    lax.fori_loop(0, nv, keep_body, (zero, zero))


_mesh = plsc.VectorSubcoreMesh(core_axis_name="c", subcore_axis_name="s")


@functools.partial(
    pl.kernel,
    out_type=jax.ShapeDtypeStruct((_ROWS * _K,), jnp.float32),
    mesh=_mesh,
    compiler_params=pltpu.CompilerParams(needs_layout_passes=False),
    scratch_types=[
        pltpu.VMEM((_RB * _N,), jnp.float32),
        pltpu.VMEM((_RB * _N,), jnp.float32),
        pltpu.VMEM((_N + 16,), jnp.float32),
        pltpu.VMEM((_RPW * _K,), jnp.float32),
        pltpu.SemaphoreType.DMA,
        pltpu.SemaphoreType.DMA,
    ],
)
def _sc_kmax(x_hbm, out_hbm, buf0, buf1, cand, outbuf, sem0, sem1):
    nc = 2
    wid = lax.axis_index("s") * nc + lax.axis_index("c")
    ebase = wid * _RPW * _N

    def start(buf, sem, batch):
        pltpu.async_copy(
            x_hbm.at[pl.ds(ebase + batch * (_RB * _N), _RB * _N)], buf, sem)

    def wait(buf, sem, batch):
        pltpu.make_async_copy(
            x_hbm.at[pl.ds(ebase + batch * (_RB * _N), _RB * _N)], buf,
            sem).wait()

    start(buf0, sem0, 0)
    start(buf1, sem1, 1)

    def outer(i, carry):
        b0 = i * 2
        wait(buf0, sem0, b0)
        for r in range(_RB):
            _process_row(buf0, r * _N, cand, outbuf, (b0 * _RB + r) * _K)

        @pl.when(b0 + 2 < _NB)
        def _():
            start(buf0, sem0, b0 + 2)

        b1 = b0 + 1
        wait(buf1, sem1, b1)
        for r in range(_RB):
            _process_row(buf1, r * _N, cand, outbuf, (b1 * _RB + r) * _K)

        @pl.when(b1 + 2 < _NB)
        def _():
            start(buf1, sem1, b1 + 2)

        return carry

    lax.fori_loop(0, _NB // 2, outer, 0)
    pltpu.sync_copy(outbuf, out_hbm.at[pl.ds(wid * (_RPW * _K), _RPW * _K)])


def kernel(x):
    b0, b1, n = x.shape
    xr = x.reshape(b0 * b1 * n)
    flat = _sc_kmax(xr)
    return flat.reshape(b0, b1, _K)


# speedup vs baseline: 18.1738x; 1.8738x over previous
import functools

import jax
import jax.numpy as jnp
from jax import lax
from jax.experimental import pallas as pl
from jax.experimental.pallas import tpu as pltpu
from jax.experimental.pallas import tpu_sc as plsc

_K = 32
_N = 8192
_NV = _N // 16
_ROWS = 16384
_NW = 32
_RPW = _ROWS // _NW
_RB = 2
_NB = _RPW // _RB

_NEG = float("-inf")


def _sort_desc(v):
    s, _ = plsc.sort_key_val(v, v, descending=True)
    return s


def _rev(v):
    return lax.rev(v, dimensions=(0,))


def _merge32_16(t0, t1, b):
    n1 = _sort_desc(jnp.maximum(t1, _rev(b)))
    hi = jnp.maximum(t0, _rev(n1))
    lo = jnp.minimum(t0, _rev(n1))
    return _sort_desc(hi), _sort_desc(lo)


def _process_row(buf, roff, cand, outbuf, out_base):
    lane = lax.iota(jnp.int32, 16)
    ninfv = jnp.full((16,), _NEG, jnp.float32)
    zero = jnp.zeros((16,), jnp.int32)
    one = jnp.full((16,), 1, jnp.int32)

    def fold_body(g, acc):
        f0, f1 = acc
        base = roff + g * 256
        xs = [buf[pl.ds(base + i * 16, 16)] for i in range(16)]

        def tree8(v):
            m01 = jnp.maximum(v[0], v[1])
            m23 = jnp.maximum(v[2], v[3])
            m45 = jnp.maximum(v[4], v[5])
            m67 = jnp.maximum(v[6], v[7])
            return jnp.maximum(jnp.maximum(m01, m23), jnp.maximum(m45, m67))

        return jnp.maximum(f0, tree8(xs[:8])), jnp.maximum(f1, tree8(xs[8:]))

    f0, f1 = lax.fori_loop(0, _N // 256, fold_body, (ninfv, ninfv))
    t = jnp.min(jnp.minimum(f0, f1))
    tv = jnp.broadcast_to(t, (16,))

    def scan_grp(g, cnt_s):
        base = roff + g * 128
        xs = [buf[pl.ds(base + i * 16, 16)] for i in range(8)]
        ms = [x >= tv for x in xs]
        anym = ms[0]
        for m in ms[1:]:
            anym = anym | m
        c = plsc.all_reduce_population_count(anym)[0]

        def slow(cnt):
            for x, m in zip(xs, ms):
                plsc.store_compressed(cand.at[pl.ds(cnt, 16)], x, mask=m)
                cnt = cnt + plsc.all_reduce_population_count(m)[0]
            return cnt

        return lax.cond(c > 0, slow, lambda cnt: cnt, cnt_s)

    n = lax.fori_loop(0, _NV // 8, scan_grp, jnp.int32(0))

    c0 = _sort_desc(cand[pl.ds(0, 16)])
    c1 = _sort_desc(cand[pl.ds(16, 16)])
    hi = jnp.maximum(c0, _rev(c1))
    lo = jnp.minimum(c0, _rev(c1))
    top = (_sort_desc(hi), _sort_desc(lo))
    nv = (n + 15) // 16

    def merge_body(vi, acc):
        b = cand[pl.ds(vi * 16, 16)]
        valid = (vi * 16 + lane) < n
        b = jnp.where(valid, b, ninfv)
        return _merge32_16(acc[0], acc[1], _sort_desc(b))

    t0, t1 = lax.fori_loop(2, nv, merge_body, top)
    T = jnp.min(t1)
    Tv = jnp.broadcast_to(T, (16,))

    def strict_body(vi, acc):
        b = cand[pl.ds(vi * 16, 16)]
        valid = (vi * 16 + lane) < n
        m = (b > Tv) & valid
        return acc + plsc.all_reduce_population_count(m)

    nstrict = lax.fori_loop(0, nv, strict_body, zero)
    needv = 32 - nstrict

    def keep_body(vi, acc):
        eqrun, keeprun = acc
        b = cand[pl.ds(vi * 16, 16)]
        valid = (vi * 16 + lane) < n
        strict = (b > Tv) & valid
        eq = (b == Tv) & valid
        eqpc = eqrun + jnp.cumsum(jnp.where(eq, one, zero))
        keep = strict | (eq & (eqpc <= needv))
        kpc = jnp.cumsum(jnp.where(keep, one, zero))
        pos = out_base + keeprun + kpc - 1
        plsc.store_scatter(outbuf, [pos], b, mask=keep)
        return (eqrun + plsc.all_reduce_population_count(eq),
                keeprun + plsc.all_reduce_population_count(keep))

    lax.fori_loop(0, nv, keep_body, (zero, zero))


_mesh = plsc.VectorSubcoreMesh(core_axis_name="c", subcore_axis_name="s")


@functools.partial(
    pl.kernel,
    out_type=jax.ShapeDtypeStruct((_ROWS * _K,), jnp.float32),
    mesh=_mesh,
    compiler_params=pltpu.CompilerParams(needs_layout_passes=False),
    scratch_types=[
        pltpu.VMEM((_RB * _N,), jnp.float32),
        pltpu.VMEM((_RB * _N,), jnp.float32),
        pltpu.VMEM((_N + 16,), jnp.float32),
        pltpu.VMEM((_RPW * _K,), jnp.float32),
        pltpu.SemaphoreType.DMA,
        pltpu.SemaphoreType.DMA,
    ],
)
def _sc_kmax(x_hbm, out_hbm, buf0, buf1, cand, outbuf, sem0, sem1):
    nc = 2
    wid = lax.axis_index("s") * nc + lax.axis_index("c")
    ebase = wid * _RPW * _N

    def start(buf, sem, batch):
        pltpu.async_copy(
            x_hbm.at[pl.ds(ebase + batch * (_RB * _N), _RB * _N)], buf, sem)

    def wait(buf, sem, batch):
        pltpu.make_async_copy(
            x_hbm.at[pl.ds(ebase + batch * (_RB * _N), _RB * _N)], buf,
            sem).wait()

    start(buf0, sem0, 0)
    start(buf1, sem1, 1)

    def outer(i, carry):
        b0 = i * 2
        wait(buf0, sem0, b0)
        for r in range(_RB):
            _process_row(buf0, r * _N, cand, outbuf, (b0 * _RB + r) * _K)

        @pl.when(b0 + 2 < _NB)
        def _():
            start(buf0, sem0, b0 + 2)

        b1 = b0 + 1
        wait(buf1, sem1, b1)
        for r in range(_RB):
            _process_row(buf1, r * _N, cand, outbuf, (b1 * _RB + r) * _K)

        @pl.when(b1 + 2 < _NB)
        def _():
            start(buf1, sem1, b1 + 2)

        return carry

    lax.fori_loop(0, _NB // 2, outer, 0)
    pltpu.sync_copy(outbuf, out_hbm.at[pl.ds(wid * (_RPW * _K), _RPW * _K)])


def kernel(x):
    b0, b1, n = x.shape
    xr = x.reshape(b0 * b1 * n)
    flat = _sc_kmax(xr)
    return flat.reshape(b0, b1, _K)
